# baseline (speedup 1.0000x reference)
import functools

import jax
import jax.numpy as jnp
from jax import lax
from jax.experimental import pallas as pl
from jax.experimental.pallas import tpu as pltpu
from jax.experimental.pallas import tpu_sc as plsc

TOKENS = 8192
DIM = 4096
NUM_EXPERTS = 64
TOPK = 8
EPSILON = 1e-06

NC = 2
NS = 16
L = 16
NW = NC * NS
EV = NUM_EXPERTS // L

CHUNKS = 4
CTOK = TOKENS // CHUNKS
BT = min(1024, CTOK)
TPW = CTOK // NW
NGRP = TPW // L
SLAB = TPW * NUM_EXPERTS


def _gate_body(x_ref, w_ref, b_ref, gate_ref):
    logits = lax.dot_general(
        w_ref[...], x_ref[...],
        dimension_numbers=(((1,), (1,)), ((), ())),
        preferred_element_type=jnp.float32,
    ) + b_ref[...]
    m = jnp.max(logits, axis=0, keepdims=True)
    e = jnp.exp(logits - m)
    g = e / jnp.sum(e, axis=0, keepdims=True)
    for j in range(BT // TPW):
        gate_ref[j, :, :] = g[:, j * TPW:(j + 1) * TPW]


def _tc_gate(x, W, b2, c):
    nb = CTOK // BT
    off = c * nb
    return pl.pallas_call(
        _gate_body,
        grid=(nb,),
        in_specs=[
            pl.BlockSpec((BT, DIM), lambda i, _o=off: (_o + i, 0)),
            pl.BlockSpec((NUM_EXPERTS, DIM), lambda i: (0, 0)),
            pl.BlockSpec((NUM_EXPERTS, 1), lambda i: (0, 0)),
        ],
        out_specs=pl.BlockSpec((BT // TPW, NUM_EXPERTS, TPW),
                               lambda i: (i, 0, 0)),
        out_shape=jax.ShapeDtypeStruct((NW, NUM_EXPERTS, TPW), jnp.float32),
    )(x, W, b2)


def _argmax_first(vals):
    idxs = [jnp.full((L,), e, jnp.int32) for e in range(len(vals))]
    vals = list(vals)
    while len(vals) > 1:
        nv, ni = [], []
        for a in range(0, len(vals), 2):
            take = vals[a + 1] > vals[a]
            nv.append(jnp.where(take, vals[a + 1], vals[a]))
            ni.append(jnp.where(take, idxs[a + 1], idxs[a]))
        vals, idxs = nv, ni
    return idxs[0]


def _route_body(gate_hbm, masked_hbm, part_hbm, buf_in, buf_out, buf_acc):
    wid = lax.axis_index("s") * NC + lax.axis_index("c")
    pltpu.sync_copy(gate_hbm.at[wid], buf_in)

    def group(g, carry):
        v = [buf_in[pl.ds(e * TPW + g * L, L)] for e in range(NUM_EXPERTS)]
        work = list(v)
        for _ in range(TOPK):
            mi = _argmax_first(work)
            work = [jnp.where(mi == e, -1.0, work[e])
                    for e in range(NUM_EXPERTS)]
        rows = g * L + lax.iota(jnp.int32, L)
        for e in range(NUM_EXPERTS):
            sel = work[e] < 0.0
            me = jnp.where(sel, v[e], 0.0)
            plsc.store_scatter(buf_out, [rows * NUM_EXPERTS + e], me)
        return carry

    lax.fori_loop(0, NGRP, group, 0)

    zero = jnp.zeros((L,), jnp.float32)

    def csum(t, acc):
        return tuple(acc[j] + buf_out[pl.ds(t * NUM_EXPERTS + L * j, L)]
                     for j in range(EV))

    acc = lax.fori_loop(0, TPW, csum, (zero,) * EV)
    for j in range(EV):
        buf_acc[pl.ds(L * j, L)] = acc[j]
    pltpu.sync_copy(buf_out, masked_hbm.at[wid])
    pltpu.sync_copy(buf_acc,
                    part_hbm.at[pl.ds(wid * NUM_EXPERTS, NUM_EXPERTS)])


def _norm_body(*refs):
    masked_refs = refs[:CHUNKS]
    part_refs = refs[CHUNKS:2 * CHUNKS]
    out_hbm = refs[2 * CHUNKS]
    buf_m, buf_p, buf_o = refs[2 * CHUNKS + 1:]
    wid = lax.axis_index("s") * NC + lax.axis_index("c")
    for c in range(CHUNKS):
        pltpu.sync_copy(part_refs[c], buf_p.at[pl.ds(c * NW * NUM_EXPERTS,
                                                     NW * NUM_EXPERTS)])
    zero = jnp.zeros((L,), jnp.float32)

    def red(w2, acc):
        return tuple(acc[j] + buf_p[pl.ds(w2 * NUM_EXPERTS + L * j, L)]
                     for j in range(EV))

    tot = lax.fori_loop(0, CHUNKS * NW, red, (zero,) * EV)
    inv = [1.0 / (tot[j] + EPSILON) for j in range(EV)]

    def row(t, carry):
        for j in range(EV):
            base = t * NUM_EXPERTS + L * j
            buf_o[pl.ds(base, L)] = buf_m[pl.ds(base, L)] * inv[j]
        return carry

    for c in range(CHUNKS):
        pltpu.sync_copy(masked_refs[c].at[wid], buf_m)
        lax.fori_loop(0, TPW, row, 0)
        pltpu.sync_copy(buf_o, out_hbm.at[c * NW + wid])


_sc_mesh = plsc.VectorSubcoreMesh(core_axis_name="c", subcore_axis_name="s")

_route = functools.partial(
    pl.kernel,
    out_type=[
        jax.ShapeDtypeStruct((NW, SLAB), jnp.float32),
        jax.ShapeDtypeStruct((NW * NUM_EXPERTS,), jnp.float32),
    ],
    mesh=_sc_mesh,
    scratch_types=[
        pltpu.VMEM((SLAB,), jnp.float32),
        pltpu.VMEM((SLAB,), jnp.float32),
        pltpu.VMEM((NUM_EXPERTS,), jnp.float32),
    ],
    compiler_params=pltpu.CompilerParams(needs_layout_passes=False),
)(_route_body)

_norm = functools.partial(
    pl.kernel,
    out_type=jax.ShapeDtypeStruct((CHUNKS * NW, SLAB), jnp.float32),
    mesh=_sc_mesh,
    scratch_types=[
        pltpu.VMEM((SLAB,), jnp.float32),
        pltpu.VMEM((CHUNKS * NW * NUM_EXPERTS,), jnp.float32),
        pltpu.VMEM((SLAB,), jnp.float32),
    ],
)(_norm_body)


@jax.jit
def kernel(x, W, b):
    b2 = b.reshape(NUM_EXPERTS, 1)
    masked, parts = [], []
    for c in range(CHUNKS):
        gate = _tc_gate(x, W, b2, c)
        m, p = _route(gate.reshape(NW, SLAB))
        masked.append(m)
        parts.append(p)
    out = _norm(*masked, *parts)
    return out.reshape(TOKENS, NUM_EXPERTS)


# speedup vs baseline: 1.1876x; 1.1876x over previous
import functools

import jax
import jax.numpy as jnp
from jax import lax
from jax.experimental import pallas as pl
from jax.experimental.pallas import tpu as pltpu
from jax.experimental.pallas import tpu_sc as plsc

TOKENS = 8192
DIM = 4096
NUM_EXPERTS = 64
TOPK = 8
EPSILON = 1e-06

NC = 2
NS = 16
L = 16
NW = NC * NS
EV = NUM_EXPERTS // L

CHUNKS = 1
CTOK = TOKENS // CHUNKS
BT = min(1024, CTOK)
TPW = CTOK // NW
NGRP = TPW // L
SLAB = TPW * NUM_EXPERTS


def _gate_body(x_ref, w_ref, b_ref, gate_ref):
    logits = lax.dot_general(
        w_ref[...], x_ref[...],
        dimension_numbers=(((1,), (1,)), ((), ())),
        preferred_element_type=jnp.float32,
    ) + b_ref[...]
    m = jnp.max(logits, axis=0, keepdims=True)
    e = jnp.exp(logits - m)
    g = e / jnp.sum(e, axis=0, keepdims=True)
    for j in range(BT // TPW):
        gate_ref[j, :, :] = g[:, j * TPW:(j + 1) * TPW]


def _tc_gate(x, W, b2, c):
    nb = CTOK // BT
    off = c * nb
    return pl.pallas_call(
        _gate_body,
        grid=(nb,),
        in_specs=[
            pl.BlockSpec((BT, DIM), lambda i, _o=off: (_o + i, 0)),
            pl.BlockSpec((NUM_EXPERTS, DIM), lambda i: (0, 0)),
            pl.BlockSpec((NUM_EXPERTS, 1), lambda i: (0, 0)),
        ],
        out_specs=pl.BlockSpec((BT // TPW, NUM_EXPERTS, TPW),
                               lambda i: (i, 0, 0)),
        out_shape=jax.ShapeDtypeStruct((NW, NUM_EXPERTS, TPW), jnp.float32),
    )(x, W, b2)


def _argmax_first(vals):
    idxs = [jnp.full((L,), e, jnp.int32) for e in range(len(vals))]
    vals = list(vals)
    while len(vals) > 1:
        nv, ni = [], []
        for a in range(0, len(vals), 2):
            take = vals[a + 1] > vals[a]
            nv.append(jnp.where(take, vals[a + 1], vals[a]))
            ni.append(jnp.where(take, idxs[a + 1], idxs[a]))
        vals, idxs = nv, ni
    return idxs[0]


def _route_body(gate_hbm, masked_hbm, part_hbm, buf_in, buf_out, buf_acc):
    wid = lax.axis_index("s") * NC + lax.axis_index("c")
    pltpu.sync_copy(gate_hbm.at[wid], buf_in)

    def group(g, carry):
        v = [buf_in[pl.ds(e * TPW + g * L, L)] for e in range(NUM_EXPERTS)]
        work = list(v)
        for _ in range(TOPK):
            t = list(work)
            while len(t) > 1:
                t = [jnp.maximum(t[a], t[a + 1]) for a in range(0, len(t), 2)]
            m = t[0]
            work = [jnp.where(work[e] == m, -1.0, work[e])
                    for e in range(NUM_EXPERTS)]
        rows = g * L + lax.iota(jnp.int32, L)
        for e in range(NUM_EXPERTS):
            sel = work[e] < 0.0
            me = jnp.where(sel, v[e], 0.0)
            plsc.store_scatter(buf_out, [rows * NUM_EXPERTS + e], me)
        return carry

    lax.fori_loop(0, NGRP, group, 0)

    zero = jnp.zeros((L,), jnp.float32)

    def csum(t, acc):
        return tuple(acc[j] + buf_out[pl.ds(t * NUM_EXPERTS + L * j, L)]
                     for j in range(EV))

    acc = lax.fori_loop(0, TPW, csum, (zero,) * EV)
    for j in range(EV):
        buf_acc[pl.ds(L * j, L)] = acc[j]
    pltpu.sync_copy(buf_out, masked_hbm.at[wid])
    pltpu.sync_copy(buf_acc,
                    part_hbm.at[pl.ds(wid * NUM_EXPERTS, NUM_EXPERTS)])


def _norm_body(*refs):
    masked_refs = refs[:CHUNKS]
    part_refs = refs[CHUNKS:2 * CHUNKS]
    out_hbm = refs[2 * CHUNKS]
    buf_m, buf_p, buf_o = refs[2 * CHUNKS + 1:]
    wid = lax.axis_index("s") * NC + lax.axis_index("c")
    for c in range(CHUNKS):
        pltpu.sync_copy(part_refs[c], buf_p.at[pl.ds(c * NW * NUM_EXPERTS,
                                                     NW * NUM_EXPERTS)])
    zero = jnp.zeros((L,), jnp.float32)

    def red(w2, acc):
        return tuple(acc[j] + buf_p[pl.ds(w2 * NUM_EXPERTS + L * j, L)]
                     for j in range(EV))

    tot = lax.fori_loop(0, CHUNKS * NW, red, (zero,) * EV)
    inv = [1.0 / (tot[j] + EPSILON) for j in range(EV)]

    def row(t, carry):
        for j in range(EV):
            base = t * NUM_EXPERTS + L * j
            buf_o[pl.ds(base, L)] = buf_m[pl.ds(base, L)] * inv[j]
        return carry

    for c in range(CHUNKS):
        pltpu.sync_copy(masked_refs[c].at[wid], buf_m)
        lax.fori_loop(0, TPW, row, 0)
        pltpu.sync_copy(buf_o, out_hbm.at[c * NW + wid])


_sc_mesh = plsc.VectorSubcoreMesh(core_axis_name="c", subcore_axis_name="s")

_route = functools.partial(
    pl.kernel,
    out_type=[
        jax.ShapeDtypeStruct((NW, SLAB), jnp.float32),
        jax.ShapeDtypeStruct((NW * NUM_EXPERTS,), jnp.float32),
    ],
    mesh=_sc_mesh,
    scratch_types=[
        pltpu.VMEM((SLAB,), jnp.float32),
        pltpu.VMEM((SLAB,), jnp.float32),
        pltpu.VMEM((NUM_EXPERTS,), jnp.float32),
    ],
    compiler_params=pltpu.CompilerParams(needs_layout_passes=False),
)(_route_body)

_norm = functools.partial(
    pl.kernel,
    out_type=jax.ShapeDtypeStruct((CHUNKS * NW, SLAB), jnp.float32),
    mesh=_sc_mesh,
    scratch_types=[
        pltpu.VMEM((SLAB,), jnp.float32),
        pltpu.VMEM((CHUNKS * NW * NUM_EXPERTS,), jnp.float32),
        pltpu.VMEM((SLAB,), jnp.float32),
    ],
)(_norm_body)


@jax.jit
def kernel(x, W, b):
    b2 = b.reshape(NUM_EXPERTS, 1)
    masked, parts = [], []
    for c in range(CHUNKS):
        gate = _tc_gate(x, W, b2, c)
        m, p = _route(gate.reshape(NW, SLAB))
        masked.append(m)
        parts.append(p)
    out = _norm(*masked, *parts)
    return out.reshape(TOKENS, NUM_EXPERTS)
